# baseline (speedup 1.0000x reference)
import jax
import jax.numpy as jnp
from jax.experimental import pallas as pl
from jax.experimental.pallas import tpu as pltpu

_N = 5000
_B = 128
_NB = 40
_NP = _NB * _B
_IOU_T = 0.3
_SCORE_T = 0.01
_WSTEP = 4 * _B


def _nms_kernel(c_ref, s_ref, out_ref, wc_ref):
    lane = jax.lax.broadcasted_iota(jnp.int32, (_B, _B), 1)
    sub = jax.lax.broadcasted_iota(jnp.int32, (_B, _B), 0)
    upper = lane > sub

    wc_ref[...] = jnp.zeros((8, _NP), jnp.float32)

    def wide_sup(width, x1c, y1c, x2c, y2c, arc):
        x1g = wc_ref[0:1, :width]
        y1g = wc_ref[1:2, :width]
        x2g = wc_ref[2:3, :width]
        y2g = wc_ref[3:4, :width]
        areag = wc_ref[4:5, :width]
        xx1 = jnp.maximum(x1c, x1g)
        yy1 = jnp.maximum(y1c, y1g)
        xx2 = jnp.minimum(x2c, x2g)
        yy2 = jnp.minimum(y2c, y2g)
        iw = jnp.maximum(xx2 - xx1, 0.0)
        ih = jnp.maximum(yy2 - yy1, 0.0)
        inter = iw * ih
        union = arc + areag - inter
        iou = inter / union
        best = jnp.max(iou, axis=1, keepdims=True)
        return jnp.where(best > _IOU_T, 1.0, 0.0)

    def block_body(i, carry):
        x1r = c_ref[0, pl.ds(i, 1), :]
        y1r = c_ref[1, pl.ds(i, 1), :]
        x2r = c_ref[2, pl.ds(i, 1), :]
        y2r = c_ref[3, pl.ds(i, 1), :]
        sr = s_ref[pl.ds(i, 1), :]
        arear = (x2r - x1r) * (y2r - y1r)
        st = jnp.concatenate([x1r, y1r, x2r, y2r, arear], axis=0).T
        x1c = st[:, 0:1]
        y1c = st[:, 1:2]
        x2c = st[:, 2:3]
        y2c = st[:, 3:4]
        arc = st[:, 4:5]

        branches = [
            (lambda w: (lambda: wide_sup(w, x1c, y1c, x2c, y2c, arc)))(
                _WSTEP * (k + 1)) for k in range(10)
        ]
        sup_col = jax.lax.switch(i // 4, branches)
        sup_row = jnp.broadcast_to(sup_col, (_B, _B)).T[0:1, :]

        xx1 = jnp.maximum(x1c, x1r)
        yy1 = jnp.maximum(y1c, y1r)
        xx2 = jnp.minimum(x2c, x2r)
        yy2 = jnp.minimum(y2c, y2r)
        iw = jnp.maximum(xx2 - xx1, 0.0)
        ih = jnp.maximum(yy2 - yy1, 0.0)
        inter = iw * ih
        union = arc + arear - inter
        iou = inter / union
        mgtu = jnp.where((iou > _IOU_T) & upper, 1.0, 0.0).astype(jnp.bfloat16)

        invalid = jnp.where(sr > _SCORE_T, 0.0, 1.0)
        act = 1.0 - jnp.maximum(sup_row, invalid)

        def fx_cond(state):
            u, l = state
            return jnp.sum(u - l) > 0.5

        def fx_step(u, l):
            mm = jax.lax.dot_general(
                jnp.concatenate([u, l], axis=0).astype(jnp.bfloat16), mgtu,
                (((1,), (0,)), ((), ())),
                preferred_element_type=jnp.float32)
            u_new = act * jnp.where(mm[1:2, :] > 0.5, 0.0, 1.0)
            l_new = act * jnp.where(mm[0:1, :] > 0.5, 0.0, 1.0)
            return u_new, l_new

        def fx_body(state):
            u, l = fx_step(*state)
            u, l = fx_step(u, l)
            u, l = fx_step(u, l)
            return fx_step(u, l)

        _, keep_b = jax.lax.while_loop(
            fx_cond, fx_body, (act, jnp.zeros((1, _B), jnp.float32)))

        out_ref[pl.ds(i, 1), :] = keep_b
        base = i * _B
        wc_ref[0:1, pl.ds(base, _B)] = x1r * keep_b
        wc_ref[1:2, pl.ds(base, _B)] = y1r * keep_b
        wc_ref[2:3, pl.ds(base, _B)] = x2r * keep_b
        wc_ref[3:4, pl.ds(base, _B)] = y2r * keep_b
        wc_ref[4:5, pl.ds(base, _B)] = arear * keep_b
        return carry

    jax.lax.fori_loop(0, _NB, block_body, 0)
    out_ref[...] = out_ref[...] * s_ref[...]


def kernel(boxes, scores):
    pad = _NP - _N
    keys = jnp.concatenate([-scores, jnp.full((pad,), 1.0, jnp.float32)])
    bt = boxes.T
    ops = [jnp.concatenate([bt[c], jnp.zeros((pad,), jnp.float32)])
           for c in range(4)]
    sk, sx1, sy1, sx2, sy2 = jax.lax.sort(
        (keys, ops[0], ops[1], ops[2], ops[3]), num_keys=1, is_stable=True)
    coords = jnp.stack([sx1, sy1, sx2, sy2]).reshape(4, _NB, _B)
    sgrid = (-sk).reshape(_NB, _B)
    b = jnp.stack([sx1, sy1, sx2, sy2], axis=1)[:_N]
    ks = pl.pallas_call(
        _nms_kernel,
        out_shape=jax.ShapeDtypeStruct((_NB, _B), jnp.float32),
        scratch_shapes=[pltpu.VMEM((8, _NP), jnp.float32)],
    )(coords, sgrid)
    return b * ks.reshape(_NP)[:_N, None]


# speedup vs baseline: 1.3467x; 1.3467x over previous
import jax
import jax.numpy as jnp
from jax.experimental import pallas as pl
from jax.experimental.pallas import tpu as pltpu

_N = 5000
_PB = 256
_NPB = 20
_NP = _NPB * _PB
_IOU_T = 0.3
_SCORE_T = 0.01
_WSTEP = 512


def _nms_kernel(c_ref, s_ref, out_ref, wc_ref):
    lane = jax.lax.broadcasted_iota(jnp.int32, (_PB, _PB), 1)
    sub = jax.lax.broadcasted_iota(jnp.int32, (_PB, _PB), 0)
    upper = lane > sub

    wc_ref[...] = jnp.zeros((8, _NP), jnp.float32)

    def wide_sup(width, x1c, y1c, x2c, y2c, arc):
        x1g = wc_ref[0:1, :width]
        y1g = wc_ref[1:2, :width]
        x2g = wc_ref[2:3, :width]
        y2g = wc_ref[3:4, :width]
        areag = wc_ref[4:5, :width]
        xx1 = jnp.maximum(x1c, x1g)
        yy1 = jnp.maximum(y1c, y1g)
        xx2 = jnp.minimum(x2c, x2g)
        yy2 = jnp.minimum(y2c, y2g)
        iw = jnp.maximum(xx2 - xx1, 0.0)
        ih = jnp.maximum(yy2 - yy1, 0.0)
        inter = iw * ih
        union = arc + areag - inter
        iou = inter / union
        best = jnp.max(iou, axis=1, keepdims=True)
        return jnp.where(best > _IOU_T, 1.0, 0.0)

    def block_body(p, carry):
        x1r = c_ref[0, pl.ds(p, 1), :]
        y1r = c_ref[1, pl.ds(p, 1), :]
        x2r = c_ref[2, pl.ds(p, 1), :]
        y2r = c_ref[3, pl.ds(p, 1), :]
        sr = s_ref[pl.ds(p, 1), :]
        arear = (x2r - x1r) * (y2r - y1r)
        st = jnp.concatenate([x1r, y1r, x2r, y2r, arear], axis=0).T
        x1c = st[:, 0:1]
        y1c = st[:, 1:2]
        x2c = st[:, 2:3]
        y2c = st[:, 3:4]
        arc = st[:, 4:5]

        branches = [
            (lambda w: (lambda: wide_sup(w, x1c, y1c, x2c, y2c, arc)))(
                _WSTEP * (k + 1)) for k in range(10)
        ]
        sup_col = jax.lax.switch(jnp.maximum(p - 1, 0) // 2, branches)
        sup_row = jnp.broadcast_to(sup_col, (_PB, _PB)).T[0:1, :]

        xx1 = jnp.maximum(x1c, x1r)
        yy1 = jnp.maximum(y1c, y1r)
        xx2 = jnp.minimum(x2c, x2r)
        yy2 = jnp.minimum(y2c, y2r)
        iw = jnp.maximum(xx2 - xx1, 0.0)
        ih = jnp.maximum(yy2 - yy1, 0.0)
        inter = iw * ih
        union = arc + arear - inter
        iou = inter / union
        mgtu = jnp.where((iou > _IOU_T) & upper, 1.0, 0.0).astype(jnp.bfloat16)

        invalid = jnp.where(sr > _SCORE_T, 0.0, 1.0)
        act = 1.0 - jnp.maximum(sup_row, invalid)

        def fx_cond(state):
            u, l = state
            return jnp.sum(u - l) > 0.5

        def fx_step(u, l):
            mm = jax.lax.dot_general(
                jnp.concatenate([u, l], axis=0).astype(jnp.bfloat16), mgtu,
                (((1,), (0,)), ((), ())),
                preferred_element_type=jnp.float32)
            u_new = act * jnp.where(mm[1:2, :] > 0.5, 0.0, 1.0)
            l_new = act * jnp.where(mm[0:1, :] > 0.5, 0.0, 1.0)
            return u_new, l_new

        def fx_body(state):
            u, l = fx_step(*state)
            return fx_step(u, l)

        _, keep_b = jax.lax.while_loop(
            fx_cond, fx_body, (act, jnp.zeros((1, _PB), jnp.float32)))

        out_ref[pl.ds(p, 1), :] = keep_b
        base = p * _PB
        wc_ref[0:1, pl.ds(base, _PB)] = x1r * keep_b
        wc_ref[1:2, pl.ds(base, _PB)] = y1r * keep_b
        wc_ref[2:3, pl.ds(base, _PB)] = x2r * keep_b
        wc_ref[3:4, pl.ds(base, _PB)] = y2r * keep_b
        wc_ref[4:5, pl.ds(base, _PB)] = arear * keep_b
        return carry

    jax.lax.fori_loop(0, _NPB, block_body, 0)
    out_ref[...] = out_ref[...] * s_ref[...]


def kernel(boxes, scores):
    pad = _NP - _N
    keys = jnp.concatenate([-scores, jnp.full((pad,), 1.0, jnp.float32)])
    bt = boxes.T
    ops = [jnp.concatenate([bt[c], jnp.zeros((pad,), jnp.float32)])
           for c in range(4)]
    sk, sx1, sy1, sx2, sy2 = jax.lax.sort(
        (keys, ops[0], ops[1], ops[2], ops[3]), num_keys=1, is_stable=True)
    coords = jnp.stack([sx1, sy1, sx2, sy2]).reshape(4, _NPB, _PB)
    sgrid = (-sk).reshape(_NPB, _PB)
    b = jnp.stack([sx1, sy1, sx2, sy2], axis=1)[:_N]
    ks = pl.pallas_call(
        _nms_kernel,
        out_shape=jax.ShapeDtypeStruct((_NPB, _PB), jnp.float32),
        scratch_shapes=[pltpu.VMEM((8, _NP), jnp.float32)],
    )(coords, sgrid)
    return b * ks.reshape(_NP)[:_N, None]
